# baseline (speedup 1.0000x reference)
import functools

import jax
import jax.numpy as jnp
from jax import lax
from jax.experimental import pallas as pl
from jax.experimental.pallas import tpu as pltpu
from jax.experimental.pallas import tpu_sc as plsc

M = 4096
R = 20
D = 1000
NCH = 5
CW = D // NCH
NC = 2
NS = 16
NW = NC * NS
S = 2
NP = NW // S
MWS = M // NP
RG = R // S
L = 16
NG = MWS // L
UPB = 2 * NCH


@functools.partial(
    pl.kernel,
    mesh=plsc.VectorSubcoreMesh(core_axis_name="c", subcore_axis_name="s"),
    compiler_params=pltpu.CompilerParams(needs_layout_passes=False),
    out_type=jax.ShapeDtypeStruct((R, D, M), jnp.float32),
    scratch_types=[
        pltpu.VMEM((RG, MWS), jnp.int32),
        pltpu.VMEM((CW, MWS), jnp.float32),
        pltpu.VMEM((CW, MWS), jnp.float32),
        pltpu.SemaphoreType.DMA,
        pltpu.SemaphoreType.DMA,
        pltpu.SemaphoreType.DMA,
    ],
)
def _onehot_sc(xt_hbm, z_hbm, out_hbm, idx_v, buf_a, buf_b, sem_a, sem_b, sem_i):
    cid = lax.axis_index("c")
    sid = lax.axis_index("s")
    wid = sid * NC + cid
    p = wid // S
    h = wid % S
    m0 = p * MWS
    r0 = h * RG

    for rr in range(RG):
        pltpu.make_async_copy(
            xt_hbm.at[r0 + rr, pl.ds(m0, MWS)], idx_v.at[rr], sem_i
        ).start()
    pltpu.sync_copy(z_hbm, buf_a)
    pltpu.sync_copy(z_hbm, buf_b)
    for rr in range(RG):
        pltpu.make_async_copy(
            xt_hbm.at[r0 + rr, pl.ds(m0, MWS)], idx_v.at[rr], sem_i
        ).wait()

    lanes = lax.iota(jnp.int32, L)
    ones = jnp.full((L,), 1.0, jnp.float32)
    zeros = jnp.zeros((L,), jnp.float32)
    bufs = (buf_a, buf_b)
    sems = (sem_a, sem_b)

    def fill(rr, lo, bi, val):
        for g in range(NG):
            cols = idx_v[rr, pl.ds(g * L, L)] - lo
            mask = (cols >= 0) & (cols < CW)
            plsc.store_scatter(bufs[bi], [cols, g * L + lanes], val, mask=mask)

    def start(r_abs, lo, bi):
        pltpu.make_async_copy(
            bufs[bi], out_hbm.at[r_abs, pl.ds(lo, CW), pl.ds(m0, MWS)], sems[bi]
        ).start()

    def wait(bi):
        pltpu.make_async_copy(
            bufs[bi], out_hbm.at[r0, pl.ds(0, CW), pl.ds(m0, MWS)], sems[bi]
        ).wait()

    for k in range(UPB):
        rr, lo, bi = k // NCH, (k % NCH) * CW, k % 2
        if k >= 2:
            kt = k - 2
            wait(bi)
            fill(kt // NCH, (kt % NCH) * CW, bi, zeros)
        fill(rr, lo, bi, ones)
        start(r0 + rr, lo, bi)

    def pair_body(it, carry):
        rb = it * 2
        for k in range(UPB):
            rr, lo, bi = rb + k // NCH, (k % NCH) * CW, k % 2
            kt = k - 2 if k >= 2 else k + UPB - 2
            rt = (rb if k >= 2 else rb - 2) + kt // NCH
            wait(bi)
            fill(rt, (kt % NCH) * CW, bi, zeros)
            fill(rr, lo, bi, ones)
            start(r0 + rr, lo, bi)
        return carry

    lax.fori_loop(1, RG // 2, pair_body, 0)
    wait(0)
    wait(1)


def kernel(x, eye):
    del eye
    xt = jnp.transpose(x)
    zeros = jnp.zeros((CW, MWS), jnp.float32)
    out = _onehot_sc(xt, zeros)
    return jnp.transpose(out, (2, 0, 1))


# speedup vs baseline: 1.0381x; 1.0381x over previous
import functools

import jax
import jax.numpy as jnp
from jax import lax
from jax.experimental import pallas as pl
from jax.experimental.pallas import tpu as pltpu
from jax.experimental.pallas import tpu_sc as plsc

M = 4096
R = 20
D = 1000
CA = 496
CB = D - CA
NC = 2
NS = 16
NW = NC * NS
S = 2
NP = NW // S
MWS = M // NP
RG = R // S
L = 16
NG = MWS // L


@functools.partial(
    pl.kernel,
    mesh=plsc.VectorSubcoreMesh(core_axis_name="c", subcore_axis_name="s"),
    compiler_params=pltpu.CompilerParams(needs_layout_passes=False),
    out_type=jax.ShapeDtypeStruct((R, D, M), jnp.float32),
    scratch_types=[
        pltpu.VMEM((MWS,), jnp.int32),
        pltpu.VMEM((CB, MWS), jnp.float32),
    ],
)
def _onehot_sc(xt_hbm, z_hbm, out_hbm, idx_v, buf_v):
    cid = lax.axis_index("c")
    sid = lax.axis_index("s")
    wid = sid * NC + cid
    p = wid // S
    h = wid % S
    m0 = p * MWS
    r0 = h * RG
    pltpu.sync_copy(z_hbm, buf_v)

    lanes = lax.iota(jnp.int32, L)
    ones = jnp.full((L,), 1.0, jnp.float32)
    zeros = jnp.zeros((L,), jnp.float32)

    def scatter(lo, hi, val):
        for g in range(NG):
            cols = idx_v[pl.ds(g * L, L)] - lo
            mask = (cols >= 0) & (cols < hi - lo)
            plsc.store_scatter(buf_v, [cols, g * L + lanes], val, mask=mask)

    def slab_body(r, carry):
        pltpu.sync_copy(xt_hbm.at[r, pl.ds(m0, MWS)], idx_v)
        for lo, hi in ((0, CA), (CA, D)):
            scatter(lo, hi, ones)
            pltpu.sync_copy(
                buf_v.at[pl.ds(0, hi - lo)],
                out_hbm.at[r, pl.ds(lo, hi - lo), pl.ds(m0, MWS)],
            )
            scatter(lo, hi, zeros)
        return carry

    lax.fori_loop(r0, r0 + RG, slab_body, 0)


def kernel(x, eye):
    del eye
    xt = jnp.transpose(x)
    zeros = jnp.zeros((CB, MWS), jnp.float32)
    out = _onehot_sc(xt, zeros)
    return jnp.transpose(out, (2, 0, 1))


# speedup vs baseline: 1.0454x; 1.0070x over previous
import functools

import jax
import jax.numpy as jnp
from jax import lax
from jax.experimental import pallas as pl
from jax.experimental.pallas import tpu as pltpu
from jax.experimental.pallas import tpu_sc as plsc

M = 4096
R = 20
D = 1000
NC = 2
NS = 16
NW = NC * NS
S = 2
NP = NW // S
MWS = M // NP
RG = R // S
L = 16
NG = MWS // L

CHUNKS = ((0, 256, 0), (256, 248, 1), (504, 248, 0), (752, 248, 1))


@functools.partial(
    pl.kernel,
    mesh=plsc.VectorSubcoreMesh(core_axis_name="c", subcore_axis_name="s"),
    compiler_params=pltpu.CompilerParams(needs_layout_passes=False),
    out_type=jax.ShapeDtypeStruct((R, D, M), jnp.float32),
    scratch_types=[
        pltpu.VMEM((2, MWS), jnp.int32),
        pltpu.VMEM((256, MWS), jnp.float32),
        pltpu.VMEM((248, MWS), jnp.float32),
        pltpu.SemaphoreType.DMA,
        pltpu.SemaphoreType.DMA,
    ],
)
def _onehot_sc(xt_hbm, z_hbm, out_hbm, idx_v, buf_a, buf_b, sem_a, sem_b):
    cid = lax.axis_index("c")
    sid = lax.axis_index("s")
    wid = sid * NC + cid
    p = wid // S
    h = wid % S
    m0 = p * MWS
    r0 = h * RG
    pltpu.sync_copy(z_hbm.at[pl.ds(0, 256)], buf_a)
    pltpu.sync_copy(z_hbm.at[pl.ds(0, 248)], buf_b)

    lanes = lax.iota(jnp.int32, L)
    ones = jnp.full((L,), 1.0, jnp.float32)
    zeros = jnp.zeros((L,), jnp.float32)
    bufs = (buf_a, buf_b)
    sems = (sem_a, sem_b)

    def load_idx(r, ip):
        pltpu.sync_copy(xt_hbm.at[r, pl.ds(m0, MWS)], idx_v.at[ip])

    def fill(ip, lo, ln, bi, val):
        for g in range(NG):
            cols = idx_v[ip, pl.ds(g * L, L)] - lo
            mask = (cols >= 0) & (cols < ln)
            plsc.store_scatter(bufs[bi], [cols, g * L + lanes], val, mask=mask)

    def start(r, lo, ln, bi):
        pltpu.make_async_copy(
            bufs[bi].at[pl.ds(0, ln)],
            out_hbm.at[r, pl.ds(lo, ln), pl.ds(m0, MWS)],
            sems[bi],
        ).start()

    def wait(ln, bi):
        pltpu.make_async_copy(
            bufs[bi].at[pl.ds(0, ln)],
            out_hbm.at[r0, pl.ds(0, ln), pl.ds(m0, MWS)],
            sems[bi],
        ).wait()

    load_idx(r0, 0)
    for k, (lo, ln, bi) in enumerate(CHUNKS):
        if k >= 2:
            plo, pln, _ = CHUNKS[k - 2]
            wait(pln, bi)
            fill(0, plo, pln, bi, zeros)
        fill(0, lo, ln, bi, ones)
        start(r0, lo, ln, bi)

    def slab_body(rr, carry):
        r = r0 + rr
        ip = rr % 2
        load_idx(r, ip)
        for k, (lo, ln, bi) in enumerate(CHUNKS):
            plo, pln, _ = CHUNKS[k - 2]
            wait(pln, bi)
            pip = ip if k >= 2 else 1 - ip
            fill(pip, plo, pln, bi, zeros)
            fill(ip, lo, ln, bi, ones)
            start(r, lo, ln, bi)
        return carry

    lax.fori_loop(1, RG, slab_body, 0)
    wait(CHUNKS[2][1], CHUNKS[2][2])
    wait(CHUNKS[3][1], CHUNKS[3][2])


def kernel(x, eye):
    del eye
    xt = jnp.transpose(x)
    zeros = jnp.zeros((256, MWS), jnp.float32)
    out = _onehot_sc(xt, zeros)
    return jnp.transpose(out, (2, 0, 1))
